# baseline (speedup 1.0000x reference)
import functools

import jax
import jax.numpy as jnp
from jax import lax
from jax.experimental import pallas as pl
from jax.experimental.pallas import tpu as pltpu
from jax.experimental.pallas import tpu_sc as plsc

D_MODEL = 768
N_EXP = 8
D_FF = 1536
TOKENS = 2048
D_RIN = 64
M = 128
NT_MAX = 40
P = NT_MAX * M
TBLK = 256


def _router_body(hr, rr, wrh_r, wrc_r, br_r,
                 pos0_r, pos1_r, g0_r, g1_r, te_r, na_r, bal_r):
    x = hr[...]
    logits = (jnp.dot(x, wrh_r[...], preferred_element_type=jnp.float32)
              + jnp.dot(rr[...], wrc_r[...], preferred_element_type=jnp.float32)
              + br_r[...])
    col = lax.broadcasted_iota(jnp.int32, (TOKENS, N_EXP), 1)
    m0 = jnp.max(logits, axis=1, keepdims=True)
    i0 = jnp.min(jnp.where(logits == m0, col, N_EXP), axis=1, keepdims=True)
    lm = jnp.where(col == i0, -jnp.inf, logits)
    m1 = jnp.max(lm, axis=1, keepdims=True)
    i1 = jnp.min(jnp.where(lm == m1, col, N_EXP), axis=1, keepdims=True)
    e1 = jnp.exp(m1 - m0)
    den = 1.0 + e1
    sel0 = (col == i0).astype(jnp.float32)
    sel1 = (col == i1).astype(jnp.float32)
    csel = sel0 + sel1

    p = jnp.exp(logits - m0)
    probs = p / jnp.sum(p, axis=1, keepdims=True)
    imp = jnp.sum(probs, axis=0, keepdims=True)
    cnt = jnp.sum(csel, axis=0, keepdims=True)
    bal_r[...] = ((4.0 / (TOKENS * TOKENS)) * jnp.sum(imp * cnt)).reshape(1, 1)

    rblk = 128
    rowi = lax.broadcasted_iota(jnp.int32, (rblk, rblk), 0)
    coli = lax.broadcasted_iota(jnp.int32, (rblk, rblk), 1)
    ltb = (coli < rowi).astype(jnp.bfloat16)
    chunks = []
    running = jnp.zeros((1, N_EXP), jnp.float32)
    for b in range(TOKENS // rblk):
        cb = lax.slice(csel, (b * rblk, 0), ((b + 1) * rblk, N_EXP))
        r = jnp.dot(ltb, cb.astype(jnp.bfloat16),
                    preferred_element_type=jnp.float32) + running
        running = running + jnp.sum(cb, axis=0, keepdims=True)
        chunks.append(r)
    ranks = jnp.concatenate(chunks, axis=0)

    cnt_i = cnt.astype(jnp.int32)
    padded = ((cnt_i + (M - 1)) >> 7) << 7
    padded_f = padded.astype(jnp.float32)
    eu = lax.broadcasted_iota(jnp.int32, (N_EXP, N_EXP), 0)
    ev = lax.broadcasted_iota(jnp.int32, (N_EXP, N_EXP), 1)
    su = (eu < ev).astype(jnp.float32)
    off = jnp.dot(padded_f, su, preferred_element_type=jnp.float32)
    off_incl = off + padded_f

    pos0_r[...] = jnp.sum(sel0 * (off + ranks), axis=1,
                          keepdims=True).astype(jnp.int32)
    pos1_r[...] = jnp.sum(sel1 * (off + ranks), axis=1,
                          keepdims=True).astype(jnp.int32)
    g0_r[...] = 1.0 / den
    g1_r[...] = e1 / den

    ti = lax.broadcasted_iota(jnp.int32, (NT_MAX, N_EXP), 0) * M
    te = jnp.sum((ti >= off_incl.astype(jnp.int32)).astype(jnp.int32),
                 axis=1, keepdims=True)
    te_r[...] = jnp.minimum(te, N_EXP - 1)
    na_r[...] = (off_incl[0:1, N_EXP - 1:N_EXP] / float(M)).astype(jnp.int32)


def _router(h, router_in, wrh, wrc, br2):
    return pl.pallas_call(
        _router_body,
        out_shape=[
            jax.ShapeDtypeStruct((TOKENS, 1), jnp.int32),
            jax.ShapeDtypeStruct((TOKENS, 1), jnp.int32),
            jax.ShapeDtypeStruct((TOKENS, 1), jnp.float32),
            jax.ShapeDtypeStruct((TOKENS, 1), jnp.float32),
            jax.ShapeDtypeStruct((NT_MAX, 1), jnp.int32),
            jax.ShapeDtypeStruct((1, 1), jnp.int32),
            jax.ShapeDtypeStruct((1, 1), jnp.float32),
        ],
    )(h, router_in, wrh, wrc, br2)


@functools.lru_cache(maxsize=1)
def _build_sc_kernels():
    info = plsc.get_sparse_core_info()
    nc, ns = info.num_cores, info.num_subcores
    nw = nc * ns
    tpw = TOKENS // nw
    mesh = plsc.VectorSubcoreMesh(core_axis_name="c", subcore_axis_name="s")

    @functools.partial(
        pl.kernel, mesh=mesh,
        out_type=jax.ShapeDtypeStruct((P, D_MODEL), jnp.float32),
        scratch_types=[
            pltpu.VMEM((tpw,), jnp.int32),
            pltpu.VMEM((tpw, D_MODEL), jnp.float32),
            pltpu.SemaphoreType.DMA,
        ],
    )
    def sc_dispatch(h_hbm, p0_hbm, p1_hbm, x_hbm, idx_v, rows_v, sem):
        wid = lax.axis_index("s") * nc + lax.axis_index("c")
        base = wid * tpw
        pltpu.sync_copy(h_hbm.at[pl.ds(base, tpw)], rows_v)
        pltpu.sync_copy(p0_hbm.at[pl.ds(base, tpw)], idx_v)
        pltpu.async_copy(rows_v, x_hbm.at[idx_v], sem).wait()
        pltpu.sync_copy(p1_hbm.at[pl.ds(base, tpw)], idx_v)
        pltpu.async_copy(rows_v, x_hbm.at[idx_v], sem).wait()

    @functools.partial(
        pl.kernel, mesh=mesh,
        out_type=[jax.ShapeDtypeStruct((TOKENS, D_MODEL), jnp.float32),
                  jax.ShapeDtypeStruct((TOKENS, D_MODEL), jnp.float32)],
        scratch_types=[
            pltpu.VMEM((tpw,), jnp.int32),
            pltpu.VMEM((tpw, D_MODEL), jnp.float32),
            pltpu.SemaphoreType.DMA,
        ],
    )
    def sc_combine(y_hbm, p0_hbm, p1_hbm, o0_hbm, o1_hbm, idx_v, rows_v, sem):
        wid = lax.axis_index("s") * nc + lax.axis_index("c")
        base = wid * tpw
        pltpu.sync_copy(p0_hbm.at[pl.ds(base, tpw)], idx_v)
        pltpu.async_copy(y_hbm.at[idx_v], rows_v, sem).wait()
        pltpu.sync_copy(rows_v, o0_hbm.at[pl.ds(base, tpw)])
        pltpu.sync_copy(p1_hbm.at[pl.ds(base, tpw)], idx_v)
        pltpu.async_copy(y_hbm.at[idx_v], rows_v, sem).wait()
        pltpu.sync_copy(rows_v, o1_hbm.at[pl.ds(base, tpw)])

    return sc_dispatch, sc_combine


def _sc_dispatch(h, p0f, p1f):
    return _build_sc_kernels()[0](h, p0f, p1f)


def _sc_combine(y_sorted, p0f, p1f):
    return _build_sc_kernels()[1](y_sorted, p0f, p1f)


def _ffn_body(te_ref, na_ref, x_ref, w1_ref, b1_ref, w2_ref, b2_ref, y_ref):
    i = pl.program_id(0)

    @pl.when(i < na_ref[0])
    def _():
        xb = x_ref[...].astype(jnp.bfloat16)
        hd = jnp.dot(xb, w1_ref[0], preferred_element_type=jnp.float32) \
            + b1_ref[0]
        hd = jax.nn.gelu(hd)
        y = jnp.dot(hd.astype(jnp.bfloat16), w2_ref[0],
                    preferred_element_type=jnp.float32) + b2_ref[0]
        y_ref[...] = y


def _ffn(tile_e, nact, x_sorted, w1b, b1, w2b, b2):
    grid_spec = pltpu.PrefetchScalarGridSpec(
        num_scalar_prefetch=2,
        grid=(NT_MAX,),
        in_specs=[
            pl.BlockSpec((M, D_MODEL), lambda i, te, na: (i, 0)),
            pl.BlockSpec((1, D_MODEL, D_FF), lambda i, te, na: (te[i], 0, 0)),
            pl.BlockSpec((1, 1, D_FF), lambda i, te, na: (te[i], 0, 0)),
            pl.BlockSpec((1, D_FF, D_MODEL), lambda i, te, na: (te[i], 0, 0)),
            pl.BlockSpec((1, 1, D_MODEL), lambda i, te, na: (te[i], 0, 0)),
        ],
        out_specs=pl.BlockSpec((M, D_MODEL), lambda i, te, na: (i, 0)),
    )
    return pl.pallas_call(
        _ffn_body, grid_spec=grid_spec,
        out_shape=jax.ShapeDtypeStruct((P, D_MODEL), jnp.float32),
    )(tile_e, nact, x_sorted, w1b, b1, w2b, b2)


def _combine_body(a_ref, b_ref, g0_ref, g1_ref, o_ref):
    o_ref[...] = g0_ref[...] * a_ref[...] + g1_ref[...] * b_ref[...]


def _combine(o0, o1, g0, g1):
    nb = TOKENS // TBLK
    return pl.pallas_call(
        _combine_body,
        grid=(nb,),
        in_specs=[
            pl.BlockSpec((TBLK, D_MODEL), lambda t: (t, 0)),
            pl.BlockSpec((TBLK, D_MODEL), lambda t: (t, 0)),
            pl.BlockSpec((TBLK, 1), lambda t: (t, 0)),
            pl.BlockSpec((TBLK, 1), lambda t: (t, 0)),
        ],
        out_specs=pl.BlockSpec((TBLK, D_MODEL), lambda t: (t, 0)),
        out_shape=jax.ShapeDtypeStruct((TOKENS, D_MODEL), jnp.float32),
    )(o0, o1, g0, g1)


def kernel(h, router_in, Wr, br, W1, b1, W2, b2):
    wrh = Wr[:D_MODEL]
    wrc = jnp.zeros((D_RIN, N_EXP), Wr.dtype).at[D_RIN - 10:].set(Wr[D_MODEL:])
    w1b = W1.astype(jnp.bfloat16)
    w2b = W2.astype(jnp.bfloat16)

    pos0, pos1, g0, g1, tile_e, nact, bal = _router(
        h, router_in, wrh, wrc, br.reshape(1, N_EXP))
    p0f = pos0.reshape(-1)
    p1f = pos1.reshape(-1)

    x_sorted = _sc_dispatch(h, p0f, p1f)
    y_sorted = _ffn(tile_e.reshape(-1), nact.reshape(-1),
                    x_sorted, w1b, b1.reshape(N_EXP, 1, D_FF),
                    w2b, b2.reshape(N_EXP, 1, D_MODEL))
    o0, o1 = _sc_combine(y_sorted, p0f, p1f)
    out = _combine(o0, o1, g0, g1)
    return out, bal.reshape(())


# speedup vs baseline: 1.1578x; 1.1578x over previous
import functools

import jax
import jax.numpy as jnp
from jax import lax
from jax.experimental import pallas as pl
from jax.experimental.pallas import tpu as pltpu
from jax.experimental.pallas import tpu_sc as plsc

D_MODEL = 768
N_EXP = 8
D_FF = 1536
TOKENS = 2048
D_RIN = 64
M = 128
NT_MAX = 40
P = NT_MAX * M
TBLK = 256


def _router_body(hr, rr, wrh_r, wrc_r, br_r,
                 pos0_r, pos1_r, g0_r, g1_r, te_r, na_r, bal_r):
    x = hr[...]
    logits = (jnp.dot(x, wrh_r[...], preferred_element_type=jnp.float32)
              + jnp.dot(rr[...], wrc_r[...], preferred_element_type=jnp.float32)
              + br_r[...])
    col = lax.broadcasted_iota(jnp.int32, (TOKENS, N_EXP), 1)
    m0 = jnp.max(logits, axis=1, keepdims=True)
    i0 = jnp.min(jnp.where(logits == m0, col, N_EXP), axis=1, keepdims=True)
    lm = jnp.where(col == i0, -jnp.inf, logits)
    m1 = jnp.max(lm, axis=1, keepdims=True)
    i1 = jnp.min(jnp.where(lm == m1, col, N_EXP), axis=1, keepdims=True)
    e1 = jnp.exp(m1 - m0)
    den = 1.0 + e1
    sel0 = (col == i0).astype(jnp.float32)
    sel1 = (col == i1).astype(jnp.float32)
    csel = sel0 + sel1

    p = jnp.exp(logits - m0)
    probs = p / jnp.sum(p, axis=1, keepdims=True)
    imp = jnp.sum(probs, axis=0, keepdims=True)
    cnt = jnp.sum(csel, axis=0, keepdims=True)
    bal_r[...] = ((4.0 / (TOKENS * TOKENS)) * jnp.sum(imp * cnt)).reshape(1, 1)

    rblk = 128
    rowi = lax.broadcasted_iota(jnp.int32, (rblk, rblk), 0)
    coli = lax.broadcasted_iota(jnp.int32, (rblk, rblk), 1)
    ltb = (coli < rowi).astype(jnp.bfloat16)
    chunks = []
    running = jnp.zeros((1, N_EXP), jnp.float32)
    for b in range(TOKENS // rblk):
        cb = lax.slice(csel, (b * rblk, 0), ((b + 1) * rblk, N_EXP))
        r = jnp.dot(ltb, cb.astype(jnp.bfloat16),
                    preferred_element_type=jnp.float32) + running
        running = running + jnp.sum(cb, axis=0, keepdims=True)
        chunks.append(r)
    ranks = jnp.concatenate(chunks, axis=0)

    cnt_i = cnt.astype(jnp.int32)
    padded = ((cnt_i + (M - 1)) >> 7) << 7
    padded_f = padded.astype(jnp.float32)
    eu = lax.broadcasted_iota(jnp.int32, (N_EXP, N_EXP), 0)
    ev = lax.broadcasted_iota(jnp.int32, (N_EXP, N_EXP), 1)
    su = (eu < ev).astype(jnp.float32)
    off = jnp.dot(padded_f, su, preferred_element_type=jnp.float32)
    off_incl = off + padded_f

    pos0_r[...] = jnp.sum(sel0 * (off + ranks), axis=1,
                          keepdims=True).astype(jnp.int32)
    pos1_r[...] = jnp.sum(sel1 * (off + ranks), axis=1,
                          keepdims=True).astype(jnp.int32)
    g0_r[...] = 1.0 / den
    g1_r[...] = e1 / den

    ti = lax.broadcasted_iota(jnp.int32, (NT_MAX, N_EXP), 0) * M
    te = jnp.sum((ti >= off_incl.astype(jnp.int32)).astype(jnp.int32),
                 axis=1, keepdims=True)
    te_r[...] = jnp.minimum(te, N_EXP - 1)
    na_r[...] = (off_incl[0:1, N_EXP - 1:N_EXP] / float(M)).astype(jnp.int32)


def _router(h, router_in, wrh, wrc, br2):
    return pl.pallas_call(
        _router_body,
        out_shape=[
            jax.ShapeDtypeStruct((TOKENS, 1), jnp.int32),
            jax.ShapeDtypeStruct((TOKENS, 1), jnp.int32),
            jax.ShapeDtypeStruct((TOKENS, 1), jnp.float32),
            jax.ShapeDtypeStruct((TOKENS, 1), jnp.float32),
            jax.ShapeDtypeStruct((NT_MAX, 1), jnp.int32),
            jax.ShapeDtypeStruct((1, 1), jnp.int32),
            jax.ShapeDtypeStruct((1, 1), jnp.float32),
        ],
    )(h, router_in, wrh, wrc, br2)


@functools.lru_cache(maxsize=1)
def _build_sc_kernels():
    info = plsc.get_sparse_core_info()
    nc, ns = info.num_cores, info.num_subcores
    nw = nc * ns
    tpw = TOKENS // nw
    mesh = plsc.VectorSubcoreMesh(core_axis_name="c", subcore_axis_name="s")

    @functools.partial(
        pl.kernel, mesh=mesh,
        out_type=jax.ShapeDtypeStruct((P, D_MODEL), jnp.float32),
        scratch_types=[
            pltpu.VMEM((tpw,), jnp.int32),
            pltpu.VMEM((tpw, D_MODEL), jnp.float32),
            pltpu.SemaphoreType.DMA,
        ],
    )
    def sc_dispatch(h_hbm, p0_hbm, p1_hbm, x_hbm, idx_v, rows_v, sem):
        wid = lax.axis_index("s") * nc + lax.axis_index("c")
        base = wid * tpw
        pltpu.sync_copy(h_hbm.at[pl.ds(base, tpw)], rows_v)
        pltpu.sync_copy(p0_hbm.at[pl.ds(base, tpw)], idx_v)
        pltpu.async_copy(rows_v, x_hbm.at[idx_v], sem).wait()
        pltpu.sync_copy(p1_hbm.at[pl.ds(base, tpw)], idx_v)
        pltpu.async_copy(rows_v, x_hbm.at[idx_v], sem).wait()

    @functools.partial(
        pl.kernel, mesh=mesh,
        out_type=[jax.ShapeDtypeStruct((TOKENS, D_MODEL), jnp.float32),
                  jax.ShapeDtypeStruct((TOKENS, D_MODEL), jnp.float32)],
        scratch_types=[
            pltpu.VMEM((tpw,), jnp.int32),
            pltpu.VMEM((tpw, D_MODEL), jnp.float32),
            pltpu.SemaphoreType.DMA,
        ],
    )
    def sc_combine(y_hbm, p0_hbm, p1_hbm, o0_hbm, o1_hbm, idx_v, rows_v, sem):
        wid = lax.axis_index("s") * nc + lax.axis_index("c")
        base = wid * tpw
        pltpu.sync_copy(p0_hbm.at[pl.ds(base, tpw)], idx_v)
        pltpu.async_copy(y_hbm.at[idx_v], rows_v, sem).wait()
        pltpu.sync_copy(rows_v, o0_hbm.at[pl.ds(base, tpw)])
        pltpu.sync_copy(p1_hbm.at[pl.ds(base, tpw)], idx_v)
        pltpu.async_copy(y_hbm.at[idx_v], rows_v, sem).wait()
        pltpu.sync_copy(rows_v, o1_hbm.at[pl.ds(base, tpw)])

    return sc_dispatch, sc_combine


def _sc_dispatch(h, p0f, p1f):
    return _build_sc_kernels()[0](h, p0f, p1f)


def _sc_combine(y_sorted, p0f, p1f):
    return _build_sc_kernels()[1](y_sorted, p0f, p1f)


def _ffn_body(te_ref, na_ref, x_ref, w1_ref, b1_ref, w2_ref, b2_ref, y_ref):
    i = pl.program_id(0)

    @pl.when(i < na_ref[0])
    def _():
        hd = jnp.dot(x_ref[...], w1_ref[0],
                     preferred_element_type=jnp.float32) + b1_ref[0]
        hd = jax.nn.gelu(hd)
        y = jnp.dot(hd, w2_ref[0],
                    preferred_element_type=jnp.float32) + b2_ref[0]
        y_ref[...] = y


def _ffn(tile_e, nact, x_sorted, w1b, b1, w2b, b2):
    grid_spec = pltpu.PrefetchScalarGridSpec(
        num_scalar_prefetch=2,
        grid=(NT_MAX,),
        in_specs=[
            pl.BlockSpec((M, D_MODEL), lambda i, te, na: (i, 0)),
            pl.BlockSpec((1, D_MODEL, D_FF), lambda i, te, na: (te[i], 0, 0)),
            pl.BlockSpec((1, 1, D_FF), lambda i, te, na: (te[i], 0, 0)),
            pl.BlockSpec((1, D_FF, D_MODEL), lambda i, te, na: (te[i], 0, 0)),
            pl.BlockSpec((1, 1, D_MODEL), lambda i, te, na: (te[i], 0, 0)),
        ],
        out_specs=pl.BlockSpec((M, D_MODEL), lambda i, te, na: (i, 0)),
    )
    return pl.pallas_call(
        _ffn_body, grid_spec=grid_spec,
        out_shape=jax.ShapeDtypeStruct((P, D_MODEL), jnp.float32),
    )(tile_e, nact, x_sorted, w1b, b1, w2b, b2)


def _combine_body(a_ref, b_ref, g0_ref, g1_ref, o_ref):
    o_ref[...] = g0_ref[...] * a_ref[...] + g1_ref[...] * b_ref[...]


def _combine(o0, o1, g0, g1):
    nb = TOKENS // TBLK
    return pl.pallas_call(
        _combine_body,
        grid=(nb,),
        in_specs=[
            pl.BlockSpec((TBLK, D_MODEL), lambda t: (t, 0)),
            pl.BlockSpec((TBLK, D_MODEL), lambda t: (t, 0)),
            pl.BlockSpec((TBLK, 1), lambda t: (t, 0)),
            pl.BlockSpec((TBLK, 1), lambda t: (t, 0)),
        ],
        out_specs=pl.BlockSpec((TBLK, D_MODEL), lambda t: (t, 0)),
        out_shape=jax.ShapeDtypeStruct((TOKENS, D_MODEL), jnp.float32),
    )(o0, o1, g0, g1)


def kernel(h, router_in, Wr, br, W1, b1, W2, b2):
    wrh = Wr[:D_MODEL]
    wrc = jnp.zeros((D_RIN, N_EXP), Wr.dtype).at[D_RIN - 10:].set(Wr[D_MODEL:])

    pos0, pos1, g0, g1, tile_e, nact, bal = _router(
        h, router_in, wrh, wrc, br.reshape(1, N_EXP))
    p0f = pos0.reshape(-1)
    p1f = pos1.reshape(-1)

    x_sorted = _sc_dispatch(h, p0f, p1f)
    y_sorted = _ffn(tile_e.reshape(-1), nact.reshape(-1),
                    x_sorted, W1, b1.reshape(N_EXP, 1, D_FF),
                    W2, b2.reshape(N_EXP, 1, D_MODEL))
    o0, o1 = _sc_combine(y_sorted, p0f, p1f)
    out = _combine(o0, o1, g0, g1)
    return out, bal.reshape(())


# speedup vs baseline: 1.1578x; 1.0000x over previous
import functools

import jax
import jax.numpy as jnp
from jax import lax
from jax.experimental import pallas as pl
from jax.experimental.pallas import tpu as pltpu
from jax.experimental.pallas import tpu_sc as plsc

D_MODEL = 768
N_EXP = 8
D_FF = 1536
TOKENS = 2048
D_RIN = 64
M = 128
NT_MAX = 40
P = NT_MAX * M
TBLK = 256


def _router_body(hr, rr, wrh_r, wrc_r, br_r,
                 pos0_r, pos1_r, g0_r, g1_r, te_r, na_r, bal_r):
    x = hr[...]
    logits = (jnp.dot(x, wrh_r[...], preferred_element_type=jnp.float32)
              + jnp.dot(rr[...], wrc_r[...], preferred_element_type=jnp.float32)
              + br_r[...])
    col = lax.broadcasted_iota(jnp.int32, (TOKENS, N_EXP), 1)
    m0 = jnp.max(logits, axis=1, keepdims=True)
    i0 = jnp.min(jnp.where(logits == m0, col, N_EXP), axis=1, keepdims=True)
    lm = jnp.where(col == i0, -jnp.inf, logits)
    m1 = jnp.max(lm, axis=1, keepdims=True)
    i1 = jnp.min(jnp.where(lm == m1, col, N_EXP), axis=1, keepdims=True)
    e1 = jnp.exp(m1 - m0)
    den = 1.0 + e1
    sel0 = (col == i0).astype(jnp.float32)
    sel1 = (col == i1).astype(jnp.float32)
    csel = sel0 + sel1

    p = jnp.exp(logits - m0)
    probs = p / jnp.sum(p, axis=1, keepdims=True)
    imp = jnp.sum(probs, axis=0, keepdims=True)
    cnt = jnp.sum(csel, axis=0, keepdims=True)
    bal_r[...] = ((4.0 / (TOKENS * TOKENS)) * jnp.sum(imp * cnt)).reshape(1, 1)

    rblk = 128
    rowi = lax.broadcasted_iota(jnp.int32, (rblk, rblk), 0)
    coli = lax.broadcasted_iota(jnp.int32, (rblk, rblk), 1)
    ltb = (coli < rowi).astype(jnp.bfloat16)
    chunks = []
    running = jnp.zeros((1, N_EXP), jnp.float32)
    for b in range(TOKENS // rblk):
        cb = lax.slice(csel, (b * rblk, 0), ((b + 1) * rblk, N_EXP))
        r = jnp.dot(ltb, cb.astype(jnp.bfloat16),
                    preferred_element_type=jnp.float32) + running
        running = running + jnp.sum(cb, axis=0, keepdims=True)
        chunks.append(r)
    ranks = jnp.concatenate(chunks, axis=0)

    cnt_i = cnt.astype(jnp.int32)
    padded = ((cnt_i + (M - 1)) >> 7) << 7
    padded_f = padded.astype(jnp.float32)
    eu = lax.broadcasted_iota(jnp.int32, (N_EXP, N_EXP), 0)
    ev = lax.broadcasted_iota(jnp.int32, (N_EXP, N_EXP), 1)
    su = (eu < ev).astype(jnp.float32)
    off = jnp.dot(padded_f, su, preferred_element_type=jnp.float32)
    off_incl = off + padded_f

    pos0_r[...] = jnp.sum(sel0 * (off + ranks), axis=1,
                          keepdims=True).astype(jnp.int32)
    pos1_r[...] = jnp.sum(sel1 * (off + ranks), axis=1,
                          keepdims=True).astype(jnp.int32)
    g0_r[...] = 1.0 / den
    g1_r[...] = e1 / den

    ti = lax.broadcasted_iota(jnp.int32, (NT_MAX, N_EXP), 0) * M
    te = jnp.sum((ti >= off_incl.astype(jnp.int32)).astype(jnp.int32),
                 axis=1, keepdims=True)
    te_r[...] = jnp.minimum(te, N_EXP - 1)
    na_r[...] = (off_incl[0:1, N_EXP - 1:N_EXP] / float(M)).astype(jnp.int32)


def _router(h, router_in, wrh, wrc, br2):
    return pl.pallas_call(
        _router_body,
        out_shape=[
            jax.ShapeDtypeStruct((TOKENS, 1), jnp.int32),
            jax.ShapeDtypeStruct((TOKENS, 1), jnp.int32),
            jax.ShapeDtypeStruct((TOKENS, 1), jnp.float32),
            jax.ShapeDtypeStruct((TOKENS, 1), jnp.float32),
            jax.ShapeDtypeStruct((NT_MAX, 1), jnp.int32),
            jax.ShapeDtypeStruct((1, 1), jnp.int32),
            jax.ShapeDtypeStruct((1, 1), jnp.float32),
        ],
    )(h, router_in, wrh, wrc, br2)


@functools.lru_cache(maxsize=1)
def _build_sc_kernels():
    info = plsc.get_sparse_core_info()
    nc, ns = info.num_cores, info.num_subcores
    nw = nc * ns
    tpw = TOKENS // nw
    mesh = plsc.VectorSubcoreMesh(core_axis_name="c", subcore_axis_name="s")

    @functools.partial(
        pl.kernel, mesh=mesh,
        out_type=jax.ShapeDtypeStruct((P, D_MODEL), jnp.float32),
        scratch_types=[
            pltpu.VMEM((tpw,), jnp.int32),
            pltpu.VMEM((tpw, D_MODEL), jnp.float32),
            pltpu.SemaphoreType.DMA,
        ],
    )
    def sc_dispatch(h_hbm, p0_hbm, p1_hbm, x_hbm, idx_v, rows_v, sem):
        wid = lax.axis_index("s") * nc + lax.axis_index("c")
        base = wid * tpw
        pltpu.sync_copy(h_hbm.at[pl.ds(base, tpw)], rows_v)
        pltpu.sync_copy(p0_hbm.at[pl.ds(base, tpw)], idx_v)
        pltpu.async_copy(rows_v, x_hbm.at[idx_v], sem).wait()
        pltpu.sync_copy(p1_hbm.at[pl.ds(base, tpw)], idx_v)
        pltpu.async_copy(rows_v, x_hbm.at[idx_v], sem).wait()

    @functools.partial(
        pl.kernel, mesh=mesh,
        out_type=[jax.ShapeDtypeStruct((TOKENS, D_MODEL), jnp.float32),
                  jax.ShapeDtypeStruct((TOKENS, D_MODEL), jnp.float32)],
        scratch_types=[
            pltpu.VMEM((tpw,), jnp.int32),
            pltpu.VMEM((tpw, D_MODEL), jnp.float32),
            pltpu.SemaphoreType.DMA,
        ],
    )
    def sc_combine(y_hbm, p0_hbm, p1_hbm, o0_hbm, o1_hbm, idx_v, rows_v, sem):
        wid = lax.axis_index("s") * nc + lax.axis_index("c")
        base = wid * tpw
        pltpu.sync_copy(p0_hbm.at[pl.ds(base, tpw)], idx_v)
        pltpu.async_copy(y_hbm.at[idx_v], rows_v, sem).wait()
        pltpu.sync_copy(rows_v, o0_hbm.at[pl.ds(base, tpw)])
        pltpu.sync_copy(p1_hbm.at[pl.ds(base, tpw)], idx_v)
        pltpu.async_copy(y_hbm.at[idx_v], rows_v, sem).wait()
        pltpu.sync_copy(rows_v, o1_hbm.at[pl.ds(base, tpw)])

    return sc_dispatch, sc_combine


def _sc_dispatch(h, p0f, p1f):
    return _build_sc_kernels()[0](h, p0f, p1f)


def _sc_combine(y_sorted, p0f, p1f):
    return _build_sc_kernels()[1](y_sorted, p0f, p1f)


def _ffn_body(te_ref, na_ref, x_ref, w1_ref, b1_ref, w2_ref, b2_ref, y_ref):
    i = pl.program_id(0)

    @pl.when(i < na_ref[0])
    def _():
        hd = jnp.dot(x_ref[...], w1_ref[0],
                     preferred_element_type=jnp.float32) + b1_ref[0]
        hd = jax.nn.gelu(hd)
        y = jnp.dot(hd, w2_ref[0],
                    preferred_element_type=jnp.float32) + b2_ref[0]
        y_ref[...] = y


def _ffn(tile_e, nact, x_sorted, w1b, b1, w2b, b2):
    grid_spec = pltpu.PrefetchScalarGridSpec(
        num_scalar_prefetch=2,
        grid=(NT_MAX,),
        in_specs=[
            pl.BlockSpec((M, D_MODEL), lambda i, te, na: (i, 0)),
            pl.BlockSpec((1, D_MODEL, D_FF), lambda i, te, na: (te[i], 0, 0)),
            pl.BlockSpec((1, 1, D_FF), lambda i, te, na: (te[i], 0, 0)),
            pl.BlockSpec((1, D_FF, D_MODEL), lambda i, te, na: (te[i], 0, 0)),
            pl.BlockSpec((1, 1, D_MODEL), lambda i, te, na: (te[i], 0, 0)),
        ],
        out_specs=pl.BlockSpec((M, D_MODEL), lambda i, te, na: (i, 0)),
    )
    return pl.pallas_call(
        _ffn_body, grid_spec=grid_spec,
        out_shape=jax.ShapeDtypeStruct((P, D_MODEL), jnp.float32),
        compiler_params=pltpu.CompilerParams(
            dimension_semantics=("parallel",)),
    )(tile_e, nact, x_sorted, w1b, b1, w2b, b2)


def _combine_body(a_ref, b_ref, g0_ref, g1_ref, o_ref):
    o_ref[...] = g0_ref[...] * a_ref[...] + g1_ref[...] * b_ref[...]


def _combine(o0, o1, g0, g1):
    nb = TOKENS // TBLK
    return pl.pallas_call(
        _combine_body,
        grid=(nb,),
        in_specs=[
            pl.BlockSpec((TBLK, D_MODEL), lambda t: (t, 0)),
            pl.BlockSpec((TBLK, D_MODEL), lambda t: (t, 0)),
            pl.BlockSpec((TBLK, 1), lambda t: (t, 0)),
            pl.BlockSpec((TBLK, 1), lambda t: (t, 0)),
        ],
        out_specs=pl.BlockSpec((TBLK, D_MODEL), lambda t: (t, 0)),
        out_shape=jax.ShapeDtypeStruct((TOKENS, D_MODEL), jnp.float32),
    )(o0, o1, g0, g1)


def kernel(h, router_in, Wr, br, W1, b1, W2, b2):
    wrh = Wr[:D_MODEL]
    wrc = jnp.zeros((D_RIN, N_EXP), Wr.dtype).at[D_RIN - 10:].set(Wr[D_MODEL:])

    pos0, pos1, g0, g1, tile_e, nact, bal = _router(
        h, router_in, wrh, wrc, br.reshape(1, N_EXP))
    p0f = pos0.reshape(-1)
    p1f = pos1.reshape(-1)

    x_sorted = _sc_dispatch(h, p0f, p1f)
    y_sorted = _ffn(tile_e.reshape(-1), nact.reshape(-1),
                    x_sorted, W1, b1.reshape(N_EXP, 1, D_FF),
                    W2, b2.reshape(N_EXP, 1, D_MODEL))
    o0, o1 = _sc_combine(y_sorted, p0f, p1f)
    out = _combine(o0, o1, g0, g1)
    return out, bal.reshape(())


# speedup vs baseline: 1.2227x; 1.0561x over previous
import functools

import jax
import jax.numpy as jnp
from jax import lax
from jax.experimental import pallas as pl
from jax.experimental.pallas import tpu as pltpu
from jax.experimental.pallas import tpu_sc as plsc

D_MODEL = 768
N_EXP = 8
D_FF = 1536
TOKENS = 2048
D_RIN = 64
M = 128
NT_MAX = 40
P = NT_MAX * M
TBLK = 256


def _router_body(hr, rr, wrh_r, wrc_r, br_r,
                 pos0_r, pos1_r, g0_r, g1_r, te_r, na_r, bal_r,
                 ws_r, ia_r, isa_r, ib_r, isb_r):
    x = hr[...]
    logits = (jnp.dot(x, wrh_r[...], preferred_element_type=jnp.float32)
              + jnp.dot(rr[...], wrc_r[...], preferred_element_type=jnp.float32)
              + br_r[...])
    col = lax.broadcasted_iota(jnp.int32, (TOKENS, N_EXP), 1)
    m0 = jnp.max(logits, axis=1, keepdims=True)
    i0 = jnp.min(jnp.where(logits == m0, col, N_EXP), axis=1, keepdims=True)
    lm = jnp.where(col == i0, -jnp.inf, logits)
    m1 = jnp.max(lm, axis=1, keepdims=True)
    i1 = jnp.min(jnp.where(lm == m1, col, N_EXP), axis=1, keepdims=True)
    e1 = jnp.exp(m1 - m0)
    den = 1.0 + e1
    sel0 = (col == i0).astype(jnp.float32)
    sel1 = (col == i1).astype(jnp.float32)
    csel = sel0 + sel1

    p = jnp.exp(logits - m0)
    probs = p / jnp.sum(p, axis=1, keepdims=True)
    imp = jnp.sum(probs, axis=0, keepdims=True)
    cnt = jnp.sum(csel, axis=0, keepdims=True)
    bal_r[...] = ((4.0 / (TOKENS * TOKENS)) * jnp.sum(imp * cnt)).reshape(1, 1)

    rblk = 128
    rowi = lax.broadcasted_iota(jnp.int32, (rblk, rblk), 0)
    coli = lax.broadcasted_iota(jnp.int32, (rblk, rblk), 1)
    ltb = (coli < rowi).astype(jnp.bfloat16)
    chunks = []
    running = jnp.zeros((1, N_EXP), jnp.float32)
    for b in range(TOKENS // rblk):
        cb = lax.slice(csel, (b * rblk, 0), ((b + 1) * rblk, N_EXP))
        r = jnp.dot(ltb, cb.astype(jnp.bfloat16),
                    preferred_element_type=jnp.float32) + running
        running = running + jnp.sum(cb, axis=0, keepdims=True)
        chunks.append(r)
    ranks = jnp.concatenate(chunks, axis=0)

    cnt_i = cnt.astype(jnp.int32)
    padded = ((cnt_i + (M - 1)) >> 7) << 7
    padded_f = padded.astype(jnp.float32)
    eu = lax.broadcasted_iota(jnp.int32, (N_EXP, N_EXP), 0)
    ev = lax.broadcasted_iota(jnp.int32, (N_EXP, N_EXP), 1)
    su = (eu < ev).astype(jnp.float32)
    off = jnp.dot(padded_f, su, preferred_element_type=jnp.float32)
    off_incl = off + padded_f

    pos0_r[...] = jnp.sum(sel0 * (off + ranks), axis=1,
                          keepdims=True).astype(jnp.int32)
    pos1_r[...] = jnp.sum(sel1 * (off + ranks), axis=1,
                          keepdims=True).astype(jnp.int32)
    g0_r[...] = 1.0 / den
    g1_r[...] = e1 / den

    ti = lax.broadcasted_iota(jnp.int32, (NT_MAX, N_EXP), 0) * M
    te = jnp.sum((ti >= off_incl.astype(jnp.int32)).astype(jnp.int32),
                 axis=1, keepdims=True)
    te_val = jnp.minimum(te, N_EXP - 1)
    te_r[...] = te_val
    na_r[...] = (off_incl[0:1, N_EXP - 1:N_EXP] / float(M)).astype(jnp.int32)

    pres = (padded > 0).astype(jnp.int32)
    erow = lax.broadcasted_iota(jnp.int32, (N_EXP, N_EXP), 0)
    ecol = lax.broadcasted_iota(jnp.int32, (N_EXP, N_EXP), 1)
    jrank = jnp.sum((erow < ecol).astype(jnp.int32)
                    * jnp.broadcast_to(pres.reshape(N_EXP, 1), (N_EXP, N_EXP)),
                    axis=0, keepdims=True)
    ind = ((jnp.broadcast_to(jrank, (N_EXP, N_EXP)) == erow)
           & (jnp.broadcast_to(pres, (N_EXP, N_EXP)) == 1)).astype(jnp.int32)
    d_of_r = jnp.sum(ind * (ecol + 1), axis=1, keepdims=True) - 1
    d_of_r_row = d_of_r.reshape(1, N_EXP)

    e_row = lax.broadcasted_iota(jnp.int32, (NT_MAX, N_EXP), 1)
    te_b = jnp.broadcast_to(te_val, (NT_MAX, N_EXP))
    j_i = jnp.sum((e_row == te_b).astype(jnp.int32)
                  * jnp.broadcast_to(jrank, (NT_MAX, N_EXP)),
                  axis=1, keepdims=True)
    row_i = lax.broadcasted_iota(jnp.int32, (NT_MAX, 1), 0)
    te_prev = jnp.concatenate([te_val[0:1], te_val[:-1]], axis=0)
    first = jnp.where(row_i == 0, 1, (te_val != te_prev).astype(jnp.int32))
    r2b = jnp.broadcast_to(j_i + 2, (NT_MAX, N_EXP))
    dvals = jnp.broadcast_to(d_of_r_row, (NT_MAX, N_EXP))
    d2ahead = jnp.sum(jnp.where(e_row == r2b, dvals + 1, 0),
                      axis=1, keepdims=True) - 1
    d1_bc = jnp.broadcast_to(d_of_r[1:2], (NT_MAX, 1))
    d2_bc = jnp.broadcast_to(d_of_r[2:3], (NT_MAX, 1))
    ws_r[...] = j_i % 3
    ia_r[...] = jnp.where(row_i == 0, d1_bc,
                          jnp.where(first == 1, d2ahead, -1))
    isa_r[...] = jnp.where(row_i == 0, 1, (j_i + 2) % 3)
    ib_r[...] = jnp.where(row_i == 0, d2_bc, -1)
    isb_r[...] = jnp.where(row_i == 0, 2, 0)


def _router(h, router_in, wrh, wrc, br2):
    return pl.pallas_call(
        _router_body,
        out_shape=[
            jax.ShapeDtypeStruct((TOKENS, 1), jnp.int32),
            jax.ShapeDtypeStruct((TOKENS, 1), jnp.int32),
            jax.ShapeDtypeStruct((TOKENS, 1), jnp.float32),
            jax.ShapeDtypeStruct((TOKENS, 1), jnp.float32),
            jax.ShapeDtypeStruct((NT_MAX, 1), jnp.int32),
            jax.ShapeDtypeStruct((1, 1), jnp.int32),
            jax.ShapeDtypeStruct((1, 1), jnp.float32),
            jax.ShapeDtypeStruct((NT_MAX, 1), jnp.int32),
            jax.ShapeDtypeStruct((NT_MAX, 1), jnp.int32),
            jax.ShapeDtypeStruct((NT_MAX, 1), jnp.int32),
            jax.ShapeDtypeStruct((NT_MAX, 1), jnp.int32),
            jax.ShapeDtypeStruct((NT_MAX, 1), jnp.int32),
        ],
    )(h, router_in, wrh, wrc, br2)


@functools.lru_cache(maxsize=1)
def _build_sc_kernels():
    info = plsc.get_sparse_core_info()
    nc, ns = info.num_cores, info.num_subcores
    nw = nc * ns
    tpw = TOKENS // nw
    mesh = plsc.VectorSubcoreMesh(core_axis_name="c", subcore_axis_name="s")

    @functools.partial(
        pl.kernel, mesh=mesh,
        out_type=jax.ShapeDtypeStruct((P, D_MODEL), jnp.float32),
        scratch_types=[
            pltpu.VMEM((tpw,), jnp.int32),
            pltpu.VMEM((tpw, D_MODEL), jnp.float32),
            pltpu.SemaphoreType.DMA,
        ],
    )
    def sc_dispatch(h_hbm, p0_hbm, p1_hbm, x_hbm, idx_v, rows_v, sem):
        wid = lax.axis_index("s") * nc + lax.axis_index("c")
        base = wid * tpw
        pltpu.sync_copy(h_hbm.at[pl.ds(base, tpw)], rows_v)
        pltpu.sync_copy(p0_hbm.at[pl.ds(base, tpw)], idx_v)
        pltpu.async_copy(rows_v, x_hbm.at[idx_v], sem).wait()
        pltpu.sync_copy(p1_hbm.at[pl.ds(base, tpw)], idx_v)
        pltpu.async_copy(rows_v, x_hbm.at[idx_v], sem).wait()

    @functools.partial(
        pl.kernel, mesh=mesh,
        out_type=[jax.ShapeDtypeStruct((TOKENS, D_MODEL), jnp.float32),
                  jax.ShapeDtypeStruct((TOKENS, D_MODEL), jnp.float32)],
        scratch_types=[
            pltpu.VMEM((tpw,), jnp.int32),
            pltpu.VMEM((tpw, D_MODEL), jnp.float32),
            pltpu.SemaphoreType.DMA,
        ],
    )
    def sc_combine(y_hbm, p0_hbm, p1_hbm, o0_hbm, o1_hbm, idx_v, rows_v, sem):
        wid = lax.axis_index("s") * nc + lax.axis_index("c")
        base = wid * tpw
        pltpu.sync_copy(p0_hbm.at[pl.ds(base, tpw)], idx_v)
        pltpu.async_copy(y_hbm.at[idx_v], rows_v, sem).wait()
        pltpu.sync_copy(rows_v, o0_hbm.at[pl.ds(base, tpw)])
        pltpu.sync_copy(p1_hbm.at[pl.ds(base, tpw)], idx_v)
        pltpu.async_copy(y_hbm.at[idx_v], rows_v, sem).wait()
        pltpu.sync_copy(rows_v, o1_hbm.at[pl.ds(base, tpw)])

    return sc_dispatch, sc_combine


def _sc_dispatch(h, p0f, p1f):
    return _build_sc_kernels()[0](h, p0f, p1f)


def _sc_combine(y_sorted, p0f, p1f):
    return _build_sc_kernels()[1](y_sorted, p0f, p1f)


def _ffn_body(te_ref, na_ref, ws_ref, ia_ref, isa_ref, ib_ref, isb_ref,
              x_ref, w1_hbm, b1_ref, w2_hbm, b2_ref, y_ref, w1v, w2v, sems):
    i = pl.program_id(0)
    na = na_ref[0]

    def issue(e_, s_):
        pltpu.make_async_copy(w1_hbm.at[e_], w1v.at[s_], sems.at[0, s_]).start()
        pltpu.make_async_copy(w2_hbm.at[e_], w2v.at[s_], sems.at[1, s_]).start()

    @pl.when(i == 0)
    def _():
        issue(te_ref[0], 0)

    ia = ia_ref[i]

    @pl.when(ia >= 0)
    def _():
        issue(ia, isa_ref[i])

    ib = ib_ref[i]

    @pl.when(ib >= 0)
    def _():
        issue(ib, isb_ref[i])

    @pl.when(i < na)
    def _():
        e = te_ref[i]
        prev = te_ref[jnp.maximum(i - 1, 0)]
        sl = ws_ref[i]

        @pl.when(jnp.logical_or(i == 0, e != prev))
        def _():
            pltpu.make_async_copy(w1_hbm.at[e], w1v.at[sl],
                                  sems.at[0, sl]).wait()
            pltpu.make_async_copy(w2_hbm.at[e], w2v.at[sl],
                                  sems.at[1, sl]).wait()

        hd = jnp.dot(x_ref[...], w1v[sl],
                     preferred_element_type=jnp.float32) + b1_ref[0]
        hd = jax.nn.gelu(hd)
        y = jnp.dot(hd, w2v[sl],
                    preferred_element_type=jnp.float32) + b2_ref[0]
        y_ref[...] = y


def _ffn(tile_e, nact, ws, ia, isa, ib, isb, x_sorted, w1, b1, w2, b2):
    grid_spec = pltpu.PrefetchScalarGridSpec(
        num_scalar_prefetch=7,
        grid=(NT_MAX,),
        in_specs=[
            pl.BlockSpec((M, D_MODEL), lambda i, *sp: (i, 0)),
            pl.BlockSpec(memory_space=pl.ANY),
            pl.BlockSpec((1, 1, D_FF), lambda i, *sp: (sp[0][i], 0, 0)),
            pl.BlockSpec(memory_space=pl.ANY),
            pl.BlockSpec((1, 1, D_MODEL), lambda i, *sp: (sp[0][i], 0, 0)),
        ],
        out_specs=pl.BlockSpec((M, D_MODEL), lambda i, *sp: (i, 0)),
        scratch_shapes=[
            pltpu.VMEM((3, D_MODEL, D_FF), jnp.float32),
            pltpu.VMEM((3, D_FF, D_MODEL), jnp.float32),
            pltpu.SemaphoreType.DMA((2, 3)),
        ],
    )
    return pl.pallas_call(
        _ffn_body, grid_spec=grid_spec,
        out_shape=jax.ShapeDtypeStruct((P, D_MODEL), jnp.float32),
        compiler_params=pltpu.CompilerParams(
            dimension_semantics=("arbitrary",)),
    )(tile_e, nact, ws, ia, isa, ib, isb, x_sorted, w1, b1, w2, b2)


def _combine_body(a_ref, b_ref, g0_ref, g1_ref, o_ref):
    o_ref[...] = g0_ref[...] * a_ref[...] + g1_ref[...] * b_ref[...]


def _combine(o0, o1, g0, g1):
    nb = TOKENS // TBLK
    return pl.pallas_call(
        _combine_body,
        grid=(nb,),
        in_specs=[
            pl.BlockSpec((TBLK, D_MODEL), lambda t: (t, 0)),
            pl.BlockSpec((TBLK, D_MODEL), lambda t: (t, 0)),
            pl.BlockSpec((TBLK, 1), lambda t: (t, 0)),
            pl.BlockSpec((TBLK, 1), lambda t: (t, 0)),
        ],
        out_specs=pl.BlockSpec((TBLK, D_MODEL), lambda t: (t, 0)),
        out_shape=jax.ShapeDtypeStruct((TOKENS, D_MODEL), jnp.float32),
    )(o0, o1, g0, g1)


def kernel(h, router_in, Wr, br, W1, b1, W2, b2):
    wrh = Wr[:D_MODEL]
    wrc = jnp.zeros((D_RIN, N_EXP), Wr.dtype).at[D_RIN - 10:].set(Wr[D_MODEL:])

    (pos0, pos1, g0, g1, tile_e, nact, bal,
     ws, ia, isa, ib, isb) = _router(
        h, router_in, wrh, wrc, br.reshape(1, N_EXP))
    p0f = pos0.reshape(-1)
    p1f = pos1.reshape(-1)

    x_sorted = _sc_dispatch(h, p0f, p1f)
    y_sorted = _ffn(tile_e.reshape(-1), nact.reshape(-1), ws.reshape(-1),
                    ia.reshape(-1), isa.reshape(-1), ib.reshape(-1),
                    isb.reshape(-1), x_sorted, W1,
                    b1.reshape(N_EXP, 1, D_FF), W2,
                    b2.reshape(N_EXP, 1, D_MODEL))
    o0, o1 = _sc_combine(y_sorted, p0f, p1f)
    out = _combine(o0, o1, g0, g1)
    return out, bal.reshape(())
